# baseline (speedup 1.0000x reference)
import functools

import jax
import jax.numpy as jnp
from jax import lax
from jax.experimental import pallas as pl
from jax.experimental.pallas import tpu as pltpu
from jax.experimental.pallas import tpu_sc as plsc

NUM_EMB = 1000000
TEXT_DIM = 64
OUT_DIM = 128
BATCH = 16384
HIST = 50

NC = 2
NS = 16
NW = NC * NS
ROWS_PER_W = BATCH // NW
K = 8
CHUNK_IDX = 2 * HIST
CHUNK_PAD = 104
GROUP_ROWS = 2 * K
GROUP_IDX = K * CHUNK_PAD
NGROUPS = ROWS_PER_W // GROUP_ROWS
NL = TEXT_DIM // 16

CBLK = 1024
GBLK = NUM_EMB // (2 * CBLK) + 1
NLINES = GBLK * CBLK
LASTB = (NUM_EMB + CBLK - 1) // CBLK - 1


def _tr_body(l_ref, r_ref, out_ref):
    out_ref[...] = jnp.concatenate(
        [jnp.transpose(l_ref[...]), jnp.transpose(r_ref[...])], axis=1)


_transpose_tc = pl.pallas_call(
    _tr_body,
    grid=(GBLK,),
    in_specs=[
        pl.BlockSpec((TEXT_DIM, CBLK),
                     lambda g: (0, jnp.minimum(2 * g, LASTB))),
        pl.BlockSpec((TEXT_DIM, CBLK),
                     lambda g: (0, jnp.minimum(2 * g + 1, LASTB))),
    ],
    out_specs=pl.BlockSpec((CBLK, 2 * TEXT_DIM), lambda g: (g, 0)),
    out_shape=jax.ShapeDtypeStruct((NLINES, 2 * TEXT_DIM), jnp.float32),
)


def _gather_pool_body(text8_hbm, table_hbm, out_hbm,
                      idx0, idx1, rows0, rows1, stage_v, sem0, sem1):
    wid = lax.axis_index("s") * NC + lax.axis_index("c")

    def fire(g, ib, rb, sem):
        pltpu.sync_copy(text8_hbm.at[pl.ds(wid * NGROUPS + g, 1)], ib)
        for c in range(GROUP_IDX // 16):
            v = ib[0, pl.ds(16 * c, 16)]
            p = ((v & -2048)
                 | ((v & 1023) << 1)
                 | ((v >> 10) & 1))
            ib[0, pl.ds(16 * c, 16)] = p
        for j in range(K):
            pltpu.async_copy(
                table_hbm.at[ib.at[0, pl.ds(CHUNK_PAD * j, CHUNK_PAD)]],
                rb.at[j], sem)

    def drain_reduce_store(g, ib, rb, sem):
        for j in range(K):
            pltpu.make_async_copy(
                table_hbm.at[ib.at[0, pl.ds(CHUNK_PAD * j, CHUNK_PAD)]],
                rb.at[j], sem).wait()
        for j in range(K):
            def red(r, accs, j=j):
                lo = tuple(accs[c] + rb[j, r, pl.ds(16 * c, 16)]
                           for c in range(NL))
                hi = tuple(accs[NL + c] + rb[j, HIST + r, pl.ds(16 * c, 16)]
                           for c in range(NL))
                return lo + hi

            zero = tuple(jnp.zeros((16,), jnp.float32) for _ in range(2 * NL))
            accs = lax.fori_loop(0, HIST, red, zero)
            for c in range(NL):
                stage_v[2 * j, pl.ds(16 * c, 16)] = accs[c]
                stage_v[2 * j + 1, pl.ds(16 * c, 16)] = accs[NL + c]
        pltpu.sync_copy(
            stage_v,
            out_hbm.at[pl.ds(wid * ROWS_PER_W + g * GROUP_ROWS, GROUP_ROWS)])

    fire(0, idx0, rows0, sem0)

    def body(t, carry):
        g = 2 * t
        fire(g + 1, idx1, rows1, sem1)
        drain_reduce_store(g, idx0, rows0, sem0)
        fire(g + 2, idx0, rows0, sem0)
        drain_reduce_store(g + 1, idx1, rows1, sem1)
        return carry

    lax.fori_loop(0, NGROUPS // 2 - 1, body, 0)
    fire(NGROUPS - 1, idx1, rows1, sem1)
    drain_reduce_store(NGROUPS - 2, idx0, rows0, sem0)
    drain_reduce_store(NGROUPS - 1, idx1, rows1, sem1)


@functools.cache
def _gather_pool():
    return pl.kernel(
        _gather_pool_body,
        out_type=jax.ShapeDtypeStruct((BATCH, TEXT_DIM), jnp.float32),
        mesh=plsc.VectorSubcoreMesh(core_axis_name="c", subcore_axis_name="s"),
        compiler_params=pltpu.CompilerParams(use_tc_tiling_on_sc=False),
        scratch_types=[
            pltpu.VMEM((1, GROUP_IDX), jnp.int32),
            pltpu.VMEM((1, GROUP_IDX), jnp.int32),
            pltpu.VMEM((K, CHUNK_PAD, TEXT_DIM), jnp.float32),
            pltpu.VMEM((K, CHUNK_PAD, TEXT_DIM), jnp.float32),
            pltpu.VMEM((GROUP_ROWS, TEXT_DIM), jnp.float32),
            pltpu.SemaphoreType.DMA,
            pltpu.SemaphoreType.DMA,
        ],
    )


TB = 1024


def _mlp_body(text_ref, summed_ref, w1_ref, b1_ref, w2_ref, b2_ref, out_ref):
    t = text_ref[...]
    counts = jnp.sum((t != 0).astype(jnp.float32), axis=1, keepdims=True)
    pooled = summed_ref[...] / jnp.maximum(counts, 1.0)
    h = jnp.maximum(
        jnp.dot(pooled, w1_ref[...], preferred_element_type=jnp.float32)
        + b1_ref[...], 0.0)
    out_ref[...] = (
        jnp.dot(h, w2_ref[...], preferred_element_type=jnp.float32)
        + b2_ref[...])


_mlp = pl.pallas_call(
    _mlp_body,
    grid=(BATCH // TB,),
    in_specs=[
        pl.BlockSpec((TB, HIST), lambda i: (i, 0)),
        pl.BlockSpec((TB, TEXT_DIM), lambda i: (i, 0)),
        pl.BlockSpec((TEXT_DIM, OUT_DIM), lambda i: (0, 0)),
        pl.BlockSpec((1, OUT_DIM), lambda i: (0, 0)),
        pl.BlockSpec((OUT_DIM, OUT_DIM), lambda i: (0, 0)),
        pl.BlockSpec((1, OUT_DIM), lambda i: (0, 0)),
    ],
    out_specs=pl.BlockSpec((TB, OUT_DIM), lambda i: (i, 0)),
    out_shape=jax.ShapeDtypeStruct((BATCH, OUT_DIM), jnp.float32),
)


def kernel(text, emb_table, W1, b1, W2, b2):
    text = text.astype(jnp.int32)
    text8 = jnp.pad(text.reshape(BATCH // 2, CHUNK_IDX),
                    ((0, 0), (0, CHUNK_PAD - CHUNK_IDX))
                    ).reshape(BATCH // GROUP_ROWS, GROUP_IDX)
    tableT = emb_table.T
    packed = _transpose_tc(tableT, tableT)
    table_lin = packed.reshape(2 * NLINES, TEXT_DIM)
    summed = _gather_pool()(text8, table_lin)
    return _mlp(text, summed, W1, b1.reshape(1, OUT_DIM),
                W2, b2.reshape(1, OUT_DIM))


# speedup vs baseline: 1.0973x; 1.0973x over previous
import functools

import jax
import jax.numpy as jnp
from jax import lax
from jax.experimental import pallas as pl
from jax.experimental.pallas import tpu as pltpu
from jax.experimental.pallas import tpu_sc as plsc

NUM_EMB = 1000000
TEXT_DIM = 64
OUT_DIM = 128
BATCH = 16384
HIST = 50

NC = 2
NS = 16
NW = NC * NS
ROWS_PER_W = BATCH // NW
K = 8
CHUNK_IDX = 2 * HIST
CHUNK_PAD = 104
GROUP_ROWS = 2 * K
GROUP_IDX = K * CHUNK_PAD
NGROUPS = ROWS_PER_W // GROUP_ROWS
NL = TEXT_DIM // 16

CBLK = 2048
GBLK = NUM_EMB // (2 * CBLK) + 1
NLINES = GBLK * CBLK
LASTB = (NUM_EMB + CBLK - 1) // CBLK - 1
SUPER = 2 * CBLK


def _tr_body(l_ref, r_ref, out_ref):
    out_ref[:, 0:TEXT_DIM] = jnp.transpose(l_ref[...])
    out_ref[:, TEXT_DIM:2 * TEXT_DIM] = jnp.transpose(r_ref[...])


_transpose_tc = pl.pallas_call(
    _tr_body,
    grid=(GBLK,),
    in_specs=[
        pl.BlockSpec((TEXT_DIM, CBLK),
                     lambda g: (0, jnp.minimum(2 * g, LASTB))),
        pl.BlockSpec((TEXT_DIM, CBLK),
                     lambda g: (0, jnp.minimum(2 * g + 1, LASTB))),
    ],
    out_specs=pl.BlockSpec((CBLK, 2 * TEXT_DIM), lambda g: (g, 0)),
    out_shape=jax.ShapeDtypeStruct((NLINES, 2 * TEXT_DIM), jnp.float32),
)


def _gather_pool_body(text8_hbm, table_hbm, out_hbm,
                      idx0, idx1, rows0, rows1, stage_v, sem0, sem1):
    wid = lax.axis_index("s") * NC + lax.axis_index("c")

    def fire(g, ib, rb, sem):
        pltpu.sync_copy(text8_hbm.at[pl.ds(wid * NGROUPS * K + g * K, K)], ib)
        for j in range(K):
            pltpu.async_copy(table_hbm.at[ib.at[j]], rb.at[j], sem)

    def drain_reduce_store(g, ib, rb, sem):
        for j in range(K):
            pltpu.make_async_copy(
                table_hbm.at[ib.at[j]], rb.at[j], sem).wait()
        for j in range(K):
            def red(r, accs, j=j):
                lo = tuple(accs[c] + rb[j, r, pl.ds(16 * c, 16)]
                           for c in range(NL))
                hi = tuple(accs[NL + c] + rb[j, HIST + r, pl.ds(16 * c, 16)]
                           for c in range(NL))
                return lo + hi

            zero = tuple(jnp.zeros((16,), jnp.float32) for _ in range(2 * NL))
            accs = lax.fori_loop(0, HIST, red, zero)
            for c in range(NL):
                stage_v[2 * j, pl.ds(16 * c, 16)] = accs[c]
                stage_v[2 * j + 1, pl.ds(16 * c, 16)] = accs[NL + c]
        pltpu.sync_copy(
            stage_v,
            out_hbm.at[pl.ds(wid * ROWS_PER_W + g * GROUP_ROWS, GROUP_ROWS)])

    fire(0, idx0, rows0, sem0)

    def body(t, carry):
        g = 2 * t
        fire(g + 1, idx1, rows1, sem1)
        drain_reduce_store(g, idx0, rows0, sem0)
        fire(g + 2, idx0, rows0, sem0)
        drain_reduce_store(g + 1, idx1, rows1, sem1)
        return carry

    lax.fori_loop(0, NGROUPS // 2 - 1, body, 0)
    fire(NGROUPS - 1, idx1, rows1, sem1)
    drain_reduce_store(NGROUPS - 2, idx0, rows0, sem0)
    drain_reduce_store(NGROUPS - 1, idx1, rows1, sem1)


@functools.cache
def _gather_pool():
    return pl.kernel(
        _gather_pool_body,
        out_type=jax.ShapeDtypeStruct((BATCH, TEXT_DIM), jnp.float32),
        mesh=plsc.VectorSubcoreMesh(core_axis_name="c", subcore_axis_name="s"),
        compiler_params=pltpu.CompilerParams(use_tc_tiling_on_sc=False),
        scratch_types=[
            pltpu.VMEM((K, CHUNK_PAD), jnp.int32),
            pltpu.VMEM((K, CHUNK_PAD), jnp.int32),
            pltpu.VMEM((K, CHUNK_PAD, TEXT_DIM), jnp.float32),
            pltpu.VMEM((K, CHUNK_PAD, TEXT_DIM), jnp.float32),
            pltpu.VMEM((GROUP_ROWS, TEXT_DIM), jnp.float32),
            pltpu.SemaphoreType.DMA,
            pltpu.SemaphoreType.DMA,
        ],
    )


TB = 1024


def _mlp_body(text_ref, summed_ref, w1_ref, b1_ref, w2_ref, b2_ref, out_ref):
    t = text_ref[...]
    counts = jnp.sum((t != 0).astype(jnp.float32), axis=1, keepdims=True)
    pooled = summed_ref[...] / jnp.maximum(counts, 1.0)
    h = jnp.maximum(
        jnp.dot(pooled, w1_ref[...], preferred_element_type=jnp.float32)
        + b1_ref[...], 0.0)
    out_ref[...] = (
        jnp.dot(h, w2_ref[...], preferred_element_type=jnp.float32)
        + b2_ref[...])


_mlp = pl.pallas_call(
    _mlp_body,
    grid=(BATCH // TB,),
    in_specs=[
        pl.BlockSpec((TB, HIST), lambda i: (i, 0)),
        pl.BlockSpec((TB, TEXT_DIM), lambda i: (i, 0)),
        pl.BlockSpec((TEXT_DIM, OUT_DIM), lambda i: (0, 0)),
        pl.BlockSpec((1, OUT_DIM), lambda i: (0, 0)),
        pl.BlockSpec((OUT_DIM, OUT_DIM), lambda i: (0, 0)),
        pl.BlockSpec((1, OUT_DIM), lambda i: (0, 0)),
    ],
    out_specs=pl.BlockSpec((TB, OUT_DIM), lambda i: (i, 0)),
    out_shape=jax.ShapeDtypeStruct((BATCH, OUT_DIM), jnp.float32),
)


def kernel(text, emb_table, W1, b1, W2, b2):
    text = text.astype(jnp.int32)
    tp = ((text & -SUPER)
          | ((text & (CBLK - 1)) << 1)
          | ((text >> 11) & 1))
    text8 = jnp.pad(tp.reshape(BATCH // 2, CHUNK_IDX),
                    ((0, 0), (0, CHUNK_PAD - CHUNK_IDX)))
    tableT = emb_table.T
    packed = _transpose_tc(tableT, tableT)
    table_lin = packed.reshape(2 * NLINES, TEXT_DIM)
    summed = _gather_pool()(text8, table_lin)
    return _mlp(text, summed, W1, b1.reshape(1, OUT_DIM),
                W2, b2.reshape(1, OUT_DIM))


# speedup vs baseline: 2.5654x; 2.3379x over previous
import functools

import jax
import jax.numpy as jnp
from jax import lax
from jax.experimental import pallas as pl
from jax.experimental.pallas import tpu as pltpu
from jax.experimental.pallas import tpu_sc as plsc

NUM_EMB = 1000000
TEXT_DIM = 64
OUT_DIM = 128
BATCH = 16384
HIST = 50

NC = 2
NS = 16
NW = NC * NS
ROWS_PER_W = BATCH // NW
K = 8
CHUNK_IDX = 2 * HIST
CHUNK_PAD = 104
GROUP_ROWS = 2 * K
GROUP_IDX = K * CHUNK_PAD
NGROUPS = ROWS_PER_W // GROUP_ROWS
NL = TEXT_DIM // 16

CBLK = 2048
GBLK = NUM_EMB // (2 * CBLK) + 1
NLINES = GBLK * CBLK
LASTB = (NUM_EMB + CBLK - 1) // CBLK - 1
SUPER = 2 * CBLK


def _tr_body(l_ref, r_ref, out_ref):
    out_ref[:, 0:TEXT_DIM] = jnp.transpose(l_ref[...])
    out_ref[:, TEXT_DIM:2 * TEXT_DIM] = jnp.transpose(r_ref[...])


_transpose_tc = pl.pallas_call(
    _tr_body,
    grid=(GBLK,),
    in_specs=[
        pl.BlockSpec((TEXT_DIM, CBLK),
                     lambda g: (0, jnp.minimum(2 * g, LASTB))),
        pl.BlockSpec((TEXT_DIM, CBLK),
                     lambda g: (0, jnp.minimum(2 * g + 1, LASTB))),
    ],
    out_specs=pl.BlockSpec((CBLK, 2 * TEXT_DIM), lambda g: (g, 0)),
    out_shape=jax.ShapeDtypeStruct((NLINES, 2 * TEXT_DIM), jnp.float32),
)


def _gather_pool_body(text8_hbm, table_hbm, out_hbm,
                      idx0, idx1, rows0, rows1, stage_v, sem0, sem1):
    wid = lax.axis_index("s") * NC + lax.axis_index("c")

    def fire(g, ib, rb, sem):
        pltpu.sync_copy(text8_hbm.at[pl.ds(wid * NGROUPS * K + g * K, K)], ib)
        for j in range(K):
            pltpu.async_copy(table_hbm.at[ib.at[j]], rb.at[j], sem)

    def drain_reduce_store(g, ib, rb, sem):
        for j in range(K):
            pltpu.make_async_copy(
                table_hbm.at[ib.at[j]], rb.at[j], sem).wait()
        for j in range(K):
            def red(r, accs, j=j):
                lo = tuple(accs[c] + rb[j, r, pl.ds(16 * c, 16)]
                           for c in range(NL))
                hi = tuple(accs[NL + c] + rb[j, HIST + r, pl.ds(16 * c, 16)]
                           for c in range(NL))
                return lo + hi

            zero = tuple(jnp.zeros((16,), jnp.float32) for _ in range(2 * NL))
            accs = lax.fori_loop(0, HIST, red, zero)
            for c in range(NL):
                stage_v[2 * j, pl.ds(16 * c, 16)] = accs[c]
                stage_v[2 * j + 1, pl.ds(16 * c, 16)] = accs[NL + c]
        pltpu.sync_copy(
            stage_v,
            out_hbm.at[pl.ds(wid * ROWS_PER_W + g * GROUP_ROWS, GROUP_ROWS)])

    fire(0, idx0, rows0, sem0)

    def body(t, carry):
        g = 2 * t
        fire(g + 1, idx1, rows1, sem1)
        drain_reduce_store(g, idx0, rows0, sem0)
        fire(g + 2, idx0, rows0, sem0)
        drain_reduce_store(g + 1, idx1, rows1, sem1)
        return carry

    lax.fori_loop(0, NGROUPS // 2 - 1, body, 0)
    fire(NGROUPS - 1, idx1, rows1, sem1)
    drain_reduce_store(NGROUPS - 2, idx0, rows0, sem0)
    drain_reduce_store(NGROUPS - 1, idx1, rows1, sem1)


@functools.cache
def _gather_pool():
    return pl.kernel(
        _gather_pool_body,
        out_type=jax.ShapeDtypeStruct((BATCH, TEXT_DIM), jnp.float32),
        mesh=plsc.VectorSubcoreMesh(core_axis_name="c", subcore_axis_name="s"),
        compiler_params=pltpu.CompilerParams(use_tc_tiling_on_sc=False),
        scratch_types=[
            pltpu.VMEM((K, CHUNK_IDX), jnp.int32),
            pltpu.VMEM((K, CHUNK_IDX), jnp.int32),
            pltpu.VMEM((K, CHUNK_IDX, TEXT_DIM), jnp.float32),
            pltpu.VMEM((K, CHUNK_IDX, TEXT_DIM), jnp.float32),
            pltpu.VMEM((GROUP_ROWS, TEXT_DIM), jnp.float32),
            pltpu.SemaphoreType.DMA,
            pltpu.SemaphoreType.DMA,
        ],
    )


TB = 1024


def _mlp_body(text_ref, summed_ref, w1_ref, b1_ref, w2_ref, b2_ref, out_ref):
    t = text_ref[...]
    counts = jnp.sum((t != 0).astype(jnp.float32), axis=1, keepdims=True)
    pooled = summed_ref[...] / jnp.maximum(counts, 1.0)
    h = jnp.maximum(
        jnp.dot(pooled, w1_ref[...], preferred_element_type=jnp.float32)
        + b1_ref[...], 0.0)
    out_ref[...] = (
        jnp.dot(h, w2_ref[...], preferred_element_type=jnp.float32)
        + b2_ref[...])


_mlp = pl.pallas_call(
    _mlp_body,
    grid=(BATCH // TB,),
    in_specs=[
        pl.BlockSpec((TB, HIST), lambda i: (i, 0)),
        pl.BlockSpec((TB, TEXT_DIM), lambda i: (i, 0)),
        pl.BlockSpec((TEXT_DIM, OUT_DIM), lambda i: (0, 0)),
        pl.BlockSpec((1, OUT_DIM), lambda i: (0, 0)),
        pl.BlockSpec((OUT_DIM, OUT_DIM), lambda i: (0, 0)),
        pl.BlockSpec((1, OUT_DIM), lambda i: (0, 0)),
    ],
    out_specs=pl.BlockSpec((TB, OUT_DIM), lambda i: (i, 0)),
    out_shape=jax.ShapeDtypeStruct((BATCH, OUT_DIM), jnp.float32),
)


def kernel(text, emb_table, W1, b1, W2, b2):
    text = text.astype(jnp.int32)
    tp = ((text & -SUPER)
          | ((text & (CBLK - 1)) << 1)
          | ((text >> 11) & 1))
    text8 = tp.reshape(BATCH // 2, CHUNK_IDX)
    tableT = emb_table.T
    packed = _transpose_tc(tableT, tableT)
    table_lin = packed.reshape(2 * NLINES, TEXT_DIM)
    summed = _gather_pool()(text8, table_lin)
    return _mlp(text, summed, W1, b1.reshape(1, OUT_DIM),
                W2, b2.reshape(1, OUT_DIM))


# speedup vs baseline: 2.9907x; 1.1658x over previous
import functools

import jax
import jax.numpy as jnp
from jax import lax
from jax.experimental import pallas as pl
from jax.experimental.pallas import tpu as pltpu
from jax.experimental.pallas import tpu_sc as plsc

NUM_EMB = 1000000
TEXT_DIM = 64
OUT_DIM = 128
BATCH = 16384
HIST = 50

NC = 2
NS = 16
NW = NC * NS
ROWS_PER_W = BATCH // NW
K = 8
CHUNK_IDX = 2 * HIST
CHUNK_PAD = 104
GROUP_ROWS = 2 * K
GROUP_IDX = K * CHUNK_PAD
NGROUPS = ROWS_PER_W // GROUP_ROWS
NL = TEXT_DIM // 16

CBLK = 4096
GBLK = NUM_EMB // (2 * CBLK) + 1
NLINES = GBLK * CBLK
LASTB = (NUM_EMB + CBLK - 1) // CBLK - 1
SUPER = 2 * CBLK


def _tr_body(l_ref, r_ref, out_ref):
    out_ref[:, 0:TEXT_DIM] = jnp.transpose(l_ref[...])
    out_ref[:, TEXT_DIM:2 * TEXT_DIM] = jnp.transpose(r_ref[...])


_transpose_tc = pl.pallas_call(
    _tr_body,
    grid=(GBLK,),
    in_specs=[
        pl.BlockSpec((TEXT_DIM, CBLK),
                     lambda g: (0, jnp.minimum(2 * g, LASTB))),
        pl.BlockSpec((TEXT_DIM, CBLK),
                     lambda g: (0, jnp.minimum(2 * g + 1, LASTB))),
    ],
    out_specs=pl.BlockSpec((CBLK, 2 * TEXT_DIM), lambda g: (g, 0)),
    out_shape=jax.ShapeDtypeStruct((NLINES, 2 * TEXT_DIM), jnp.float32),
)


def _gather_pool_body(text8_hbm, table_hbm, out_hbm,
                      idx0, idx1, rows0, rows1, stage_v, sem0, sem1):
    wid = lax.axis_index("s") * NC + lax.axis_index("c")

    def fire(g, ib, rb, sem):
        pltpu.sync_copy(text8_hbm.at[pl.ds(wid * NGROUPS * K + g * K, K)], ib)
        for j in range(K):
            pltpu.async_copy(table_hbm.at[ib.at[j]], rb.at[j], sem)

    def drain_reduce_store(g, ib, rb, sem):
        for j in range(K):
            pltpu.make_async_copy(
                table_hbm.at[ib.at[j]], rb.at[j], sem).wait()
        for j in range(K):
            def red(r, accs, j=j):
                lo = tuple(accs[c] + rb[j, r, pl.ds(16 * c, 16)]
                           for c in range(NL))
                hi = tuple(accs[NL + c] + rb[j, HIST + r, pl.ds(16 * c, 16)]
                           for c in range(NL))
                return lo + hi

            zero = tuple(jnp.zeros((16,), jnp.float32) for _ in range(2 * NL))
            accs = lax.fori_loop(0, HIST, red, zero)
            for c in range(NL):
                stage_v[2 * j, pl.ds(16 * c, 16)] = accs[c]
                stage_v[2 * j + 1, pl.ds(16 * c, 16)] = accs[NL + c]
        pltpu.sync_copy(
            stage_v,
            out_hbm.at[pl.ds(wid * ROWS_PER_W + g * GROUP_ROWS, GROUP_ROWS)])

    fire(0, idx0, rows0, sem0)

    def body(t, carry):
        g = 2 * t
        fire(g + 1, idx1, rows1, sem1)
        drain_reduce_store(g, idx0, rows0, sem0)
        fire(g + 2, idx0, rows0, sem0)
        drain_reduce_store(g + 1, idx1, rows1, sem1)
        return carry

    lax.fori_loop(0, NGROUPS // 2 - 1, body, 0)
    fire(NGROUPS - 1, idx1, rows1, sem1)
    drain_reduce_store(NGROUPS - 2, idx0, rows0, sem0)
    drain_reduce_store(NGROUPS - 1, idx1, rows1, sem1)


@functools.cache
def _gather_pool():
    return pl.kernel(
        _gather_pool_body,
        out_type=jax.ShapeDtypeStruct((BATCH, TEXT_DIM), jnp.float32),
        mesh=plsc.VectorSubcoreMesh(core_axis_name="c", subcore_axis_name="s"),
        compiler_params=pltpu.CompilerParams(use_tc_tiling_on_sc=False),
        scratch_types=[
            pltpu.VMEM((K, CHUNK_IDX), jnp.int32),
            pltpu.VMEM((K, CHUNK_IDX), jnp.int32),
            pltpu.VMEM((K, CHUNK_IDX, TEXT_DIM), jnp.float32),
            pltpu.VMEM((K, CHUNK_IDX, TEXT_DIM), jnp.float32),
            pltpu.VMEM((GROUP_ROWS, TEXT_DIM), jnp.float32),
            pltpu.SemaphoreType.DMA,
            pltpu.SemaphoreType.DMA,
        ],
    )


TB = 1024


def _mlp_body(text_ref, summed_ref, w1_ref, b1_ref, w2_ref, b2_ref, out_ref):
    t = text_ref[...]
    counts = jnp.sum((t != 0).astype(jnp.float32), axis=1, keepdims=True)
    pooled = summed_ref[...] / jnp.maximum(counts, 1.0)
    h = jnp.maximum(
        jnp.dot(pooled, w1_ref[...], preferred_element_type=jnp.float32)
        + b1_ref[...], 0.0)
    out_ref[...] = (
        jnp.dot(h, w2_ref[...], preferred_element_type=jnp.float32)
        + b2_ref[...])


_mlp = pl.pallas_call(
    _mlp_body,
    grid=(BATCH // TB,),
    in_specs=[
        pl.BlockSpec((TB, HIST), lambda i: (i, 0)),
        pl.BlockSpec((TB, TEXT_DIM), lambda i: (i, 0)),
        pl.BlockSpec((TEXT_DIM, OUT_DIM), lambda i: (0, 0)),
        pl.BlockSpec((1, OUT_DIM), lambda i: (0, 0)),
        pl.BlockSpec((OUT_DIM, OUT_DIM), lambda i: (0, 0)),
        pl.BlockSpec((1, OUT_DIM), lambda i: (0, 0)),
    ],
    out_specs=pl.BlockSpec((TB, OUT_DIM), lambda i: (i, 0)),
    out_shape=jax.ShapeDtypeStruct((BATCH, OUT_DIM), jnp.float32),
)


def kernel(text, emb_table, W1, b1, W2, b2):
    text = text.astype(jnp.int32)
    tp = ((text & -SUPER)
          | ((text & (CBLK - 1)) << 1)
          | ((text >> 12) & 1))
    text8 = tp.reshape(BATCH // 2, CHUNK_IDX)
    tableT = emb_table.T
    packed = _transpose_tc(tableT, tableT)
    table_lin = packed.reshape(2 * NLINES, TEXT_DIM)
    summed = _gather_pool()(text8, table_lin)
    return _mlp(text, summed, W1, b1.reshape(1, OUT_DIM),
                W2, b2.reshape(1, OUT_DIM))


# speedup vs baseline: 3.2387x; 1.0829x over previous
import functools

import jax
import jax.numpy as jnp
from jax import lax
from jax.experimental import pallas as pl
from jax.experimental.pallas import tpu as pltpu
from jax.experimental.pallas import tpu_sc as plsc

NUM_EMB = 1000000
TEXT_DIM = 64
OUT_DIM = 128
BATCH = 16384
HIST = 50

NC = 2
NS = 16
NW = NC * NS
ROWS_PER_W = BATCH // NW
K = 8
CHUNK_IDX = 2 * HIST
CHUNK_PAD = 104
GROUP_ROWS = 2 * K
GROUP_IDX = K * CHUNK_PAD
NGROUPS = ROWS_PER_W // GROUP_ROWS
NL = TEXT_DIM // 16

CBLK = 8192
GBLK = NUM_EMB // (2 * CBLK) + 1
NLINES = GBLK * CBLK
LASTB = (NUM_EMB + CBLK - 1) // CBLK - 1
SUPER = 2 * CBLK


def _tr_body(l_ref, r_ref, out_ref):
    out_ref[:, 0:TEXT_DIM] = jnp.transpose(l_ref[...])
    out_ref[:, TEXT_DIM:2 * TEXT_DIM] = jnp.transpose(r_ref[...])


_transpose_tc = pl.pallas_call(
    _tr_body,
    grid=(GBLK,),
    in_specs=[
        pl.BlockSpec((TEXT_DIM, CBLK),
                     lambda g: (0, jnp.minimum(2 * g, LASTB))),
        pl.BlockSpec((TEXT_DIM, CBLK),
                     lambda g: (0, jnp.minimum(2 * g + 1, LASTB))),
    ],
    out_specs=pl.BlockSpec((CBLK, 2 * TEXT_DIM), lambda g: (g, 0)),
    out_shape=jax.ShapeDtypeStruct((NLINES, 2 * TEXT_DIM), jnp.float32),
)


def _gather_pool_body(text8_hbm, table_hbm, out_hbm,
                      idx0, idx1, rows0, rows1, stage_v, sem0, sem1):
    wid = lax.axis_index("s") * NC + lax.axis_index("c")

    def fire(g, ib, rb, sem):
        pltpu.sync_copy(text8_hbm.at[pl.ds(wid * NGROUPS * K + g * K, K)], ib)
        for j in range(K):
            pltpu.async_copy(table_hbm.at[ib.at[j]], rb.at[j], sem)

    def drain_reduce_store(g, ib, rb, sem):
        for j in range(K):
            pltpu.make_async_copy(
                table_hbm.at[ib.at[j]], rb.at[j], sem).wait()
        for j in range(K):
            def red(r, accs, j=j):
                lo = tuple(accs[c] + rb[j, r, pl.ds(16 * c, 16)]
                           for c in range(NL))
                hi = tuple(accs[NL + c] + rb[j, HIST + r, pl.ds(16 * c, 16)]
                           for c in range(NL))
                return lo + hi

            zero = tuple(jnp.zeros((16,), jnp.float32) for _ in range(2 * NL))
            accs = lax.fori_loop(0, HIST, red, zero)
            for c in range(NL):
                stage_v[2 * j, pl.ds(16 * c, 16)] = accs[c]
                stage_v[2 * j + 1, pl.ds(16 * c, 16)] = accs[NL + c]
        pltpu.sync_copy(
            stage_v,
            out_hbm.at[pl.ds(wid * ROWS_PER_W + g * GROUP_ROWS, GROUP_ROWS)])

    fire(0, idx0, rows0, sem0)

    def body(t, carry):
        g = 2 * t
        fire(g + 1, idx1, rows1, sem1)
        drain_reduce_store(g, idx0, rows0, sem0)
        fire(g + 2, idx0, rows0, sem0)
        drain_reduce_store(g + 1, idx1, rows1, sem1)
        return carry

    lax.fori_loop(0, NGROUPS // 2 - 1, body, 0)
    fire(NGROUPS - 1, idx1, rows1, sem1)
    drain_reduce_store(NGROUPS - 2, idx0, rows0, sem0)
    drain_reduce_store(NGROUPS - 1, idx1, rows1, sem1)


@functools.cache
def _gather_pool():
    return pl.kernel(
        _gather_pool_body,
        out_type=jax.ShapeDtypeStruct((BATCH, TEXT_DIM), jnp.float32),
        mesh=plsc.VectorSubcoreMesh(core_axis_name="c", subcore_axis_name="s"),
        compiler_params=pltpu.CompilerParams(use_tc_tiling_on_sc=False),
        scratch_types=[
            pltpu.VMEM((K, CHUNK_IDX), jnp.int32),
            pltpu.VMEM((K, CHUNK_IDX), jnp.int32),
            pltpu.VMEM((K, CHUNK_IDX, TEXT_DIM), jnp.float32),
            pltpu.VMEM((K, CHUNK_IDX, TEXT_DIM), jnp.float32),
            pltpu.VMEM((GROUP_ROWS, TEXT_DIM), jnp.float32),
            pltpu.SemaphoreType.DMA,
            pltpu.SemaphoreType.DMA,
        ],
    )


TB = 1024


def _mlp_body(text_ref, summed_ref, w1_ref, b1_ref, w2_ref, b2_ref, out_ref):
    t = text_ref[...]
    counts = jnp.sum((t != 0).astype(jnp.float32), axis=1, keepdims=True)
    pooled = summed_ref[...] / jnp.maximum(counts, 1.0)
    h = jnp.maximum(
        jnp.dot(pooled, w1_ref[...], preferred_element_type=jnp.float32)
        + b1_ref[...], 0.0)
    out_ref[...] = (
        jnp.dot(h, w2_ref[...], preferred_element_type=jnp.float32)
        + b2_ref[...])


_mlp = pl.pallas_call(
    _mlp_body,
    grid=(BATCH // TB,),
    in_specs=[
        pl.BlockSpec((TB, HIST), lambda i: (i, 0)),
        pl.BlockSpec((TB, TEXT_DIM), lambda i: (i, 0)),
        pl.BlockSpec((TEXT_DIM, OUT_DIM), lambda i: (0, 0)),
        pl.BlockSpec((1, OUT_DIM), lambda i: (0, 0)),
        pl.BlockSpec((OUT_DIM, OUT_DIM), lambda i: (0, 0)),
        pl.BlockSpec((1, OUT_DIM), lambda i: (0, 0)),
    ],
    out_specs=pl.BlockSpec((TB, OUT_DIM), lambda i: (i, 0)),
    out_shape=jax.ShapeDtypeStruct((BATCH, OUT_DIM), jnp.float32),
)


def kernel(text, emb_table, W1, b1, W2, b2):
    text = text.astype(jnp.int32)
    tp = ((text & -SUPER)
          | ((text & (CBLK - 1)) << 1)
          | ((text >> 13) & 1))
    text8 = tp.reshape(BATCH // 2, CHUNK_IDX)
    tableT = emb_table.T
    packed = _transpose_tc(tableT, tableT)
    table_lin = packed.reshape(2 * NLINES, TEXT_DIM)
    summed = _gather_pool()(text8, table_lin)
    return _mlp(text, summed, W1, b1.reshape(1, OUT_DIM),
                W2, b2.reshape(1, OUT_DIM))


# speedup vs baseline: 3.3628x; 1.0383x over previous
import functools

import jax
import jax.numpy as jnp
from jax import lax
from jax.experimental import pallas as pl
from jax.experimental.pallas import tpu as pltpu
from jax.experimental.pallas import tpu_sc as plsc

NUM_EMB = 1000000
TEXT_DIM = 64
OUT_DIM = 128
BATCH = 16384
HIST = 50

NC = 2
NS = 16
NW = NC * NS
ROWS_PER_W = BATCH // NW
K = 8
CHUNK_IDX = 2 * HIST
CHUNK_PAD = 104
GROUP_ROWS = 2 * K
GROUP_IDX = K * CHUNK_PAD
NGROUPS = ROWS_PER_W // GROUP_ROWS
NL = TEXT_DIM // 16

CBLK = 16384
GBLK = NUM_EMB // (2 * CBLK) + 1
NLINES = GBLK * CBLK
LASTB = (NUM_EMB + CBLK - 1) // CBLK - 1
SUPER = 2 * CBLK


def _tr_body(l_ref, r_ref, out_ref):
    out_ref[:, 0:TEXT_DIM] = jnp.transpose(l_ref[...])
    out_ref[:, TEXT_DIM:2 * TEXT_DIM] = jnp.transpose(r_ref[...])


_transpose_tc = pl.pallas_call(
    _tr_body,
    grid=(GBLK,),
    in_specs=[
        pl.BlockSpec((TEXT_DIM, CBLK),
                     lambda g: (0, jnp.minimum(2 * g, LASTB))),
        pl.BlockSpec((TEXT_DIM, CBLK),
                     lambda g: (0, jnp.minimum(2 * g + 1, LASTB))),
    ],
    out_specs=pl.BlockSpec((CBLK, 2 * TEXT_DIM), lambda g: (g, 0)),
    out_shape=jax.ShapeDtypeStruct((NLINES, 2 * TEXT_DIM), jnp.float32),
)


def _gather_pool_body(text8_hbm, table_hbm, out_hbm,
                      idx0, idx1, rows0, rows1, stage_v, sem0, sem1):
    wid = lax.axis_index("s") * NC + lax.axis_index("c")

    def fire(g, ib, rb, sem):
        pltpu.sync_copy(text8_hbm.at[pl.ds(wid * NGROUPS * K + g * K, K)], ib)
        for j in range(K):
            pltpu.async_copy(table_hbm.at[ib.at[j]], rb.at[j], sem)

    def drain_reduce_store(g, ib, rb, sem):
        for j in range(K):
            pltpu.make_async_copy(
                table_hbm.at[ib.at[j]], rb.at[j], sem).wait()
        for j in range(K):
            def red(r, accs, j=j):
                lo = tuple(accs[c] + rb[j, r, pl.ds(16 * c, 16)]
                           for c in range(NL))
                hi = tuple(accs[NL + c] + rb[j, HIST + r, pl.ds(16 * c, 16)]
                           for c in range(NL))
                return lo + hi

            zero = tuple(jnp.zeros((16,), jnp.float32) for _ in range(2 * NL))
            accs = lax.fori_loop(0, HIST, red, zero)
            for c in range(NL):
                stage_v[2 * j, pl.ds(16 * c, 16)] = accs[c]
                stage_v[2 * j + 1, pl.ds(16 * c, 16)] = accs[NL + c]
        pltpu.sync_copy(
            stage_v,
            out_hbm.at[pl.ds(wid * ROWS_PER_W + g * GROUP_ROWS, GROUP_ROWS)])

    fire(0, idx0, rows0, sem0)

    def body(t, carry):
        g = 2 * t
        fire(g + 1, idx1, rows1, sem1)
        drain_reduce_store(g, idx0, rows0, sem0)
        fire(g + 2, idx0, rows0, sem0)
        drain_reduce_store(g + 1, idx1, rows1, sem1)
        return carry

    lax.fori_loop(0, NGROUPS // 2 - 1, body, 0)
    fire(NGROUPS - 1, idx1, rows1, sem1)
    drain_reduce_store(NGROUPS - 2, idx0, rows0, sem0)
    drain_reduce_store(NGROUPS - 1, idx1, rows1, sem1)


@functools.cache
def _gather_pool():
    return pl.kernel(
        _gather_pool_body,
        out_type=jax.ShapeDtypeStruct((BATCH, TEXT_DIM), jnp.float32),
        mesh=plsc.VectorSubcoreMesh(core_axis_name="c", subcore_axis_name="s"),
        compiler_params=pltpu.CompilerParams(use_tc_tiling_on_sc=False),
        scratch_types=[
            pltpu.VMEM((K, CHUNK_IDX), jnp.int32),
            pltpu.VMEM((K, CHUNK_IDX), jnp.int32),
            pltpu.VMEM((K, CHUNK_IDX, TEXT_DIM), jnp.float32),
            pltpu.VMEM((K, CHUNK_IDX, TEXT_DIM), jnp.float32),
            pltpu.VMEM((GROUP_ROWS, TEXT_DIM), jnp.float32),
            pltpu.SemaphoreType.DMA,
            pltpu.SemaphoreType.DMA,
        ],
    )


TB = 1024


def _mlp_body(text_ref, summed_ref, w1_ref, b1_ref, w2_ref, b2_ref, out_ref):
    t = text_ref[...]
    counts = jnp.sum((t != 0).astype(jnp.float32), axis=1, keepdims=True)
    pooled = summed_ref[...] / jnp.maximum(counts, 1.0)
    h = jnp.maximum(
        jnp.dot(pooled, w1_ref[...], preferred_element_type=jnp.float32)
        + b1_ref[...], 0.0)
    out_ref[...] = (
        jnp.dot(h, w2_ref[...], preferred_element_type=jnp.float32)
        + b2_ref[...])


_mlp = pl.pallas_call(
    _mlp_body,
    grid=(BATCH // TB,),
    in_specs=[
        pl.BlockSpec((TB, HIST), lambda i: (i, 0)),
        pl.BlockSpec((TB, TEXT_DIM), lambda i: (i, 0)),
        pl.BlockSpec((TEXT_DIM, OUT_DIM), lambda i: (0, 0)),
        pl.BlockSpec((1, OUT_DIM), lambda i: (0, 0)),
        pl.BlockSpec((OUT_DIM, OUT_DIM), lambda i: (0, 0)),
        pl.BlockSpec((1, OUT_DIM), lambda i: (0, 0)),
    ],
    out_specs=pl.BlockSpec((TB, OUT_DIM), lambda i: (i, 0)),
    out_shape=jax.ShapeDtypeStruct((BATCH, OUT_DIM), jnp.float32),
)


def kernel(text, emb_table, W1, b1, W2, b2):
    text = text.astype(jnp.int32)
    tp = ((text & -SUPER)
          | ((text & (CBLK - 1)) << 1)
          | ((text >> 14) & 1))
    text8 = tp.reshape(BATCH // 2, CHUNK_IDX)
    tableT = emb_table.T
    packed = _transpose_tc(tableT, tableT)
    table_lin = packed.reshape(2 * NLINES, TEXT_DIM)
    summed = _gather_pool()(text8, table_lin)
    return _mlp(text, summed, W1, b1.reshape(1, OUT_DIM),
                W2, b2.reshape(1, OUT_DIM))
